# baseline (speedup 1.0000x reference)
import functools

import jax
import jax.numpy as jnp
from jax import lax
from jax.experimental import pallas as pl
from jax.experimental.pallas import tpu as pltpu
from jax.experimental.pallas import tpu_sc as plsc

VOCAB = 16
D_MODEL = 8
LANES = 128
SUBL = 8


def _make_sc_kernel(B: int, T: int, chunk: int, n_workers: int, lanes: int):
  n_tokens = B * T
  assert n_tokens % (n_workers * chunk) == 0 and T % chunk == 0
  assert chunk % LANES == 0
  per_worker = n_tokens // n_workers
  rows_per_worker = per_worker // T
  assert rows_per_worker * T == per_worker
  chunks_per_row = T // chunk
  groups = chunk // lanes
  half_words = chunk * SUBL
  b_words = T * VOCAB
  v8_words = T * SUBL

  mesh = plsc.VectorSubcoreMesh(core_axis_name="c", subcore_axis_name="s")
  num_cores = mesh.num_cores

  @functools.partial(
      pl.kernel,
      out_type=jax.ShapeDtypeStruct((n_tokens * VOCAB,), jnp.float32),
      mesh=mesh,
      compiler_params=pltpu.CompilerParams(needs_layout_passes=False),
      scratch_types=[
          pltpu.VMEM((2 * VOCAB * D_MODEL + VOCAB,), jnp.float32),
          pltpu.VMEM((VOCAB * VOCAB,), jnp.float32),
          pltpu.VMEM((chunk,), jnp.int32),
          pltpu.VMEM((chunk,), jnp.int32),
          pltpu.VMEM((chunk * VOCAB,), jnp.float32),
          pltpu.VMEM((chunk * VOCAB,), jnp.float32),
          pltpu.SemaphoreType.DMA,
          pltpu.SemaphoreType.DMA,
          pltpu.SemaphoreType.DMA,
          pltpu.SemaphoreType.DMA,
          pltpu.SemaphoreType.DMA,
      ],
  )
  def sc_kernel(x_hbm, e_hbm, w_hbm, b_hbm, out_hbm, ew_v, tt_v,
                idx0, idx1, out0, out1, isem0, isem1, osem0, osem1, wsem):
    wid = lax.axis_index("s") * num_cores + lax.axis_index("c")
    row0 = wid * rows_per_worker

    idx_bufs, out_bufs = [idx0, idx1], [out0, out1]
    isems, osems = [isem0, isem1], [osem0, osem1]

    def idx_copy(c, buf):
      row = row0 + c // chunks_per_row
      col = (c % chunks_per_row) * chunk
      return pltpu.make_async_copy(
          x_hbm.at[row, pl.ds(col, chunk)], idx_bufs[buf], isems[buf])

    idx_copy(0, 0).start()

    stage = [
        pltpu.make_async_copy(
            e_hbm, ew_v.at[pl.ds(0, VOCAB * D_MODEL)], wsem),
        pltpu.make_async_copy(
            w_hbm, ew_v.at[pl.ds(VOCAB * D_MODEL, VOCAB * D_MODEL)], wsem),
        pltpu.make_async_copy(
            b_hbm, ew_v.at[pl.ds(2 * VOCAB * D_MODEL, VOCAB)], wsem),
    ]
    for d in stage:
      d.start()
    for d in stage:
      d.wait()

    lane16 = lax.iota(jnp.int32, lanes)
    e_cols = [plsc.load_gather(ew_v, [lane16 * D_MODEL + d])
              for d in range(D_MODEL)]
    for u in range(VOCAB):
      acc = plsc.load_gather(
          ew_v, [jnp.full((lanes,), 2 * VOCAB * D_MODEL + u, jnp.int32)])
      for d in range(D_MODEL):
        wbc = plsc.load_gather(
            ew_v,
            [jnp.full((lanes,), VOCAB * D_MODEL + u * D_MODEL + d,
                      jnp.int32)])
        acc = acc + e_cols[d] * wbc
      tt_v[pl.ds(u * VOCAB, VOCAB)] = acc

    def out_copy(c, buf, v8):
      row = row0 + c // chunks_per_row
      col = (c % chunks_per_row) * chunk
      off = row * b_words + v8 * v8_words + col * SUBL
      return pltpu.make_async_copy(
          out_bufs[buf].at[pl.ds(v8 * half_words, half_words)],
          out_hbm.at[pl.ds(off, half_words)], osems[buf])

    def compute(ibuf, obuf):
      @plsc.parallel_loop(0, groups, unroll=4)
      def _(g):
        idxv = ibuf[pl.ds(g * lanes, lanes)]
        gbase = (g // SUBL) * (SUBL * LANES) + (g % SUBL) * lanes
        for v in range(VOCAB):
          vals = plsc.load_gather(tt_v, [idxv + (v * VOCAB)])
          voff = (v // SUBL) * half_words + (v % SUBL) * LANES
          obuf[pl.ds(gbase + voff, lanes)] = vals

    n_chunks = rows_per_worker * chunks_per_row
    for c in range(n_chunks):
      buf = c % 2
      if c + 1 < n_chunks:
        idx_copy(c + 1, 1 - buf).start()
      idx_copy(c, buf).wait()
      if c >= 2:
        out_copy(c - 2, buf, 0).wait()
        out_copy(c - 2, buf, 1).wait()
      compute(idx_bufs[buf], out_bufs[buf])
      out_copy(c, buf, 0).start()
      out_copy(c, buf, 1).start()
    for c in (n_chunks - 2, n_chunks - 1):
      out_copy(c, c % 2, 0).wait()
      out_copy(c, c % 2, 1).wait()

  return sc_kernel


def kernel(x, embed, W, b):
  B, T = x.shape
  info = plsc.get_sparse_core_info()
  n_workers = info.num_cores * info.num_subcores
  sc_kernel = _make_sc_kernel(B, T, 2048, n_workers, info.num_lanes)
  out = sc_kernel(x, embed.reshape(VOCAB * D_MODEL), W.reshape(VOCAB * D_MODEL),
                  b)
  out = out.reshape(B, VOCAB // SUBL, T // LANES, SUBL, LANES)
  return out.transpose(0, 2, 4, 1, 3).reshape(B, T, VOCAB)


# speedup vs baseline: 1.0457x; 1.0457x over previous
import functools

import jax
import jax.numpy as jnp
from jax import lax
from jax.experimental import pallas as pl
from jax.experimental.pallas import tpu as pltpu
from jax.experimental.pallas import tpu_sc as plsc

VOCAB = 16
D_MODEL = 8
LANES = 128
SUBL = 8


def _make_sc_kernel(B: int, T: int, chunk: int, n_workers: int, lanes: int):
  n_tokens = B * T
  assert n_tokens % (n_workers * chunk) == 0 and T % chunk == 0
  assert chunk % LANES == 0
  per_worker = n_tokens // n_workers
  rows_per_worker = per_worker // T
  assert rows_per_worker * T == per_worker
  chunks_per_row = T // chunk
  groups = chunk // lanes
  half_words = chunk * SUBL
  b_words = T * VOCAB
  v8_words = T * SUBL

  mesh = plsc.VectorSubcoreMesh(core_axis_name="c", subcore_axis_name="s")
  num_cores = mesh.num_cores

  @functools.partial(
      pl.kernel,
      out_type=jax.ShapeDtypeStruct((n_tokens * VOCAB,), jnp.float32),
      mesh=mesh,
      compiler_params=pltpu.CompilerParams(needs_layout_passes=False),
      scratch_types=[
          pltpu.VMEM((2 * VOCAB * D_MODEL + VOCAB,), jnp.float32),
          pltpu.VMEM((VOCAB * VOCAB,), jnp.float32),
          pltpu.VMEM((chunk,), jnp.int32),
          pltpu.VMEM((chunk,), jnp.int32),
          pltpu.VMEM((chunk * VOCAB,), jnp.float32),
          pltpu.VMEM((chunk * VOCAB,), jnp.float32),
          pltpu.VMEM((chunk * VOCAB,), jnp.float32),
          pltpu.SemaphoreType.DMA,
          pltpu.SemaphoreType.DMA,
          pltpu.SemaphoreType.DMA,
          pltpu.SemaphoreType.DMA,
          pltpu.SemaphoreType.DMA,
          pltpu.SemaphoreType.DMA,
      ],
  )
  def sc_kernel(x_hbm, e_hbm, w_hbm, b_hbm, out_hbm, ew_v, tt_v,
                idx0, idx1, out0, out1, out2,
                isem0, isem1, osem0, osem1, osem2, wsem):
    wid = lax.axis_index("s") * num_cores + lax.axis_index("c")
    row0 = wid * rows_per_worker

    idx_bufs, out_bufs = [idx0, idx1], [out0, out1, out2]
    isems, osems = [isem0, isem1], [osem0, osem1, osem2]

    def idx_copy(c, buf):
      row = row0 + c // chunks_per_row
      col = (c % chunks_per_row) * chunk
      return pltpu.make_async_copy(
          x_hbm.at[row, pl.ds(col, chunk)], idx_bufs[buf], isems[buf])

    idx_copy(0, 0).start()

    stage = [
        pltpu.make_async_copy(
            e_hbm, ew_v.at[pl.ds(0, VOCAB * D_MODEL)], wsem),
        pltpu.make_async_copy(
            w_hbm, ew_v.at[pl.ds(VOCAB * D_MODEL, VOCAB * D_MODEL)], wsem),
        pltpu.make_async_copy(
            b_hbm, ew_v.at[pl.ds(2 * VOCAB * D_MODEL, VOCAB)], wsem),
    ]
    for d in stage:
      d.start()
    for d in stage:
      d.wait()

    lane16 = lax.iota(jnp.int32, lanes)
    e_cols = [plsc.load_gather(ew_v, [lane16 * D_MODEL + d])
              for d in range(D_MODEL)]
    for u in range(VOCAB):
      acc = plsc.load_gather(
          ew_v, [jnp.full((lanes,), 2 * VOCAB * D_MODEL + u, jnp.int32)])
      for d in range(D_MODEL):
        wbc = plsc.load_gather(
            ew_v,
            [jnp.full((lanes,), VOCAB * D_MODEL + u * D_MODEL + d,
                      jnp.int32)])
        acc = acc + e_cols[d] * wbc
      tt_v[pl.ds(u * VOCAB, VOCAB)] = acc

    def out_copy(c, buf, v8):
      row = row0 + c // chunks_per_row
      col = (c % chunks_per_row) * chunk
      off = row * b_words + v8 * v8_words + col * SUBL
      return pltpu.make_async_copy(
          out_bufs[buf].at[pl.ds(v8 * half_words, half_words)],
          out_hbm.at[pl.ds(off, half_words)], osems[buf])

    def compute(ibuf, obuf):
      @plsc.parallel_loop(0, groups, unroll=2)
      def _(g):
        idxv = ibuf[pl.ds(g * lanes, lanes)]
        gbase = (g // SUBL) * (SUBL * LANES) + (g % SUBL) * lanes
        for v in range(VOCAB):
          vals = plsc.load_gather(tt_v, [idxv + (v * VOCAB)])
          voff = (v // SUBL) * half_words + (v % SUBL) * LANES
          obuf[pl.ds(gbase + voff, lanes)] = vals

    n_chunks = rows_per_worker * chunks_per_row
    for c in range(n_chunks):
      ibuf, obuf = c % 2, c % 3
      if c + 1 < n_chunks:
        idx_copy(c + 1, 1 - ibuf).start()
      idx_copy(c, ibuf).wait()
      if c >= 3:
        out_copy(c - 3, obuf, 0).wait()
        out_copy(c - 3, obuf, 1).wait()
      compute(idx_bufs[ibuf], out_bufs[obuf])
      out_copy(c, obuf, 0).start()
      out_copy(c, obuf, 1).start()
    for c in (n_chunks - 3, n_chunks - 2, n_chunks - 1):
      out_copy(c, c % 3, 0).wait()
      out_copy(c, c % 3, 1).wait()

  return sc_kernel


def kernel(x, embed, W, b):
  B, T = x.shape
  info = plsc.get_sparse_core_info()
  n_workers = info.num_cores * info.num_subcores
  sc_kernel = _make_sc_kernel(B, T, 2048, n_workers, info.num_lanes)
  out = sc_kernel(x, embed.reshape(VOCAB * D_MODEL), W.reshape(VOCAB * D_MODEL),
                  b)
  out = out.reshape(B, VOCAB // SUBL, T // LANES, SUBL, LANES)
  return out.transpose(0, 2, 4, 1, 3).reshape(B, T, VOCAB)
